# baseline (speedup 1.0000x reference)
import jax
import jax.numpy as jnp
from jax.experimental import pallas as pl


def _copy_body(x_ref, o_ref):
    o_ref[...] = x_ref[...]


def kernel(sort_ip):
    idx = jnp.argsort(-sort_ip, axis=-1)
    return pl.pallas_call(
        _copy_body,
        out_shape=jax.ShapeDtypeStruct(idx.shape, idx.dtype),
    )(idx)


# speedup vs baseline: 2.6212x; 2.6212x over previous
import jax
import jax.numpy as jnp
from jax import lax
from jax.experimental import pallas as pl
from jax.experimental.pallas import tpu as pltpu
from jax.experimental.pallas import tpu_sc as plsc

NR = 64
N0 = 100000
L = 16
NT = 16
NC = 2
CHUNK = 6272
NP = NT * CHUNK
NW = CHUNK // 128
NV = CHUNK // L
RADIX = 256
NPASS = 4
ROWS_PER_SC = NR // NC


def _sort_body(x_hbm, out_hbm, kA, iA, kB, iB, hgrid,
               rawf, kch, ich, posg, offs, hloc, gloc, sem):
    c = lax.axis_index("c")
    t = lax.axis_index("s")
    iota = lax.iota(jnp.int32, L)
    base = t * CHUNK
    zero16 = jnp.zeros((L,), jnp.int32)

    def digit(v, sh):
        return lax.shift_right_logical(v, jnp.int32(sh)) & jnp.int32(0xFF)

    def hist_phase(p, r, kIN, iIN):
        sh = 8 * p
        if p == 0:
            pltpu.sync_copy(x_hbm.at[r, pl.ds(base, CHUNK)], rawf)

            def tl(j, _):
                b = plsc.bitcast(rawf[pl.ds(j * L, L)], jnp.int32)
                m = lax.shift_right_arithmetic(b, 31)
                kd = (b ^ (m | jnp.int32(-2147483648))) ^ jnp.int32(-1)
                kch[pl.ds(j * L, L)] = kd
                ich[pl.ds(j * L, L)] = base + j * L + iota
                return 0

            lax.fori_loop(0, NV, tl, 0)
        else:
            pltpu.sync_copy(kIN.at[pl.ds(base, CHUNK)], kch)
            pltpu.sync_copy(iIN.at[pl.ds(base, CHUNK)], ich)

        def z(j, _):
            hloc[pl.ds(j * L, L)] = zero16
            return 0

        lax.fori_loop(0, RADIX // L, z, 0)

        def hb(j, _):
            d = digit(kch[pl.ds(j * L, L)], sh)
            occ, lastm = plsc.scan_count(d)
            plsc.addupdate_scatter(hloc, [d], occ, mask=lastm)
            return 0

        lax.fori_loop(0, NV, hb, 0)
        pltpu.sync_copy(hloc, hgrid.at[t])
        plsc.subcore_barrier()

    def scan_phase():
        pltpu.sync_copy(hgrid, gloc)

        def g_body(g, runv):
            def t_body(tp, carry):
                part, tot = carry
                rowv = gloc[tp, pl.ds(g * L, L)]
                part = part + jnp.where(tp < t, rowv, zero16)
                tot = tot + rowv
                return part, tot

            part, tot = lax.fori_loop(0, NT, t_body, (zero16, zero16))
            csum = plsc.cumsum(tot)
            offs[pl.ds(g * L, L)] = runv + (csum - tot) + part
            return runv + jnp.full((L,), jnp.sum(tot), jnp.int32)

        lax.fori_loop(0, RADIX // L, g_body, zero16)

    def permute_phase(p, kOUT, iOUT):
        sh = 8 * p

        def pb(j, _):
            v = kch[pl.ds(j * L, L)]
            d = digit(v, sh)
            occ, lastm = plsc.scan_count(d)
            bse = plsc.load_gather(offs, [d])
            plsc.store_scatter(offs, [d], bse + occ, mask=lastm)
            posg[j // 8, pl.ds((j % 8) * L, L)] = bse + occ - 1
            return 0

        lax.fori_loop(0, NV, pb, 0)
        handles = []
        for w in range(NW):
            if kOUT is not None:
                handles.append(pltpu.async_copy(
                    kch.at[pl.ds(w * 128, 128)], kOUT.at[posg.at[w]], sem))
            handles.append(pltpu.async_copy(
                ich.at[pl.ds(w * 128, 128)], iOUT.at[posg.at[w]], sem))
        for h in handles:
            h.wait()
        plsc.subcore_barrier()

    def row_body(rr, _):
        r = rr * NC + c
        hist_phase(0, r, None, None)
        scan_phase()
        permute_phase(0, kA, iA)
        hist_phase(1, r, kA, iA)
        scan_phase()
        permute_phase(1, kB, iB)
        hist_phase(2, r, kB, iB)
        scan_phase()
        permute_phase(2, kA, iA)
        hist_phase(3, r, kA, iA)
        scan_phase()
        permute_phase(3, None, iB)
        pltpu.sync_copy(iB.at[pl.ds(base, CHUNK)],
                        out_hbm.at[r, pl.ds(base, CHUNK)])
        plsc.subcore_barrier()
        return 0

    lax.fori_loop(0, ROWS_PER_SC, row_body, 0)


def kernel(sort_ip):
    xp = jnp.pad(sort_ip, ((0, 0), (0, NP - N0)),
                 constant_values=-jnp.inf)
    mesh = plsc.VectorSubcoreMesh(core_axis_name="c", subcore_axis_name="s")
    fn = pl.kernel(
        _sort_body,
        out_type=jax.ShapeDtypeStruct((NR, NP), jnp.int32),
        mesh=mesh,
        compiler_params=pltpu.CompilerParams(needs_layout_passes=False),
        scratch_types=(
            pltpu.VMEM_SHARED((NP,), jnp.int32),
            pltpu.VMEM_SHARED((NP,), jnp.int32),
            pltpu.VMEM_SHARED((NP,), jnp.int32),
            pltpu.VMEM_SHARED((NP,), jnp.int32),
            pltpu.VMEM_SHARED((NT, RADIX), jnp.int32),
            pltpu.VMEM((CHUNK,), jnp.float32),
            pltpu.VMEM((CHUNK,), jnp.int32),
            pltpu.VMEM((CHUNK,), jnp.int32),
            pltpu.VMEM((NW, 128), jnp.int32),
            pltpu.VMEM((RADIX,), jnp.int32),
            pltpu.VMEM((RADIX,), jnp.int32),
            pltpu.VMEM((NT, RADIX), jnp.int32),
            pltpu.SemaphoreType.DMA,
        ),
    )
    out = fn(xp)
    return out[:, :N0]


# speedup vs baseline: 2.7258x; 1.0399x over previous
import jax
import jax.numpy as jnp
from jax import lax
from jax.experimental import pallas as pl
from jax.experimental.pallas import tpu as pltpu
from jax.experimental.pallas import tpu_sc as plsc

NR = 64
N0 = 100000
L = 16
NT = 16
NC = 2
CHUNK = 6272
NP = NT * CHUNK
NW = CHUNK // 128
NV = CHUNK // L
RADIX = 256
NPASS = 4
ROWS_PER_SC = NR // NC


def _sort_body(x_hbm, out_hbm, kA, iA, kB, iB, hgrid,
               rawf, kch, ich, posg, offs, hloc, gloc, sem):
    c = lax.axis_index("c")
    t = lax.axis_index("s")
    iota = lax.iota(jnp.int32, L)
    base = t * CHUNK
    zero16 = jnp.zeros((L,), jnp.int32)

    def digit(v, sh):
        return lax.shift_right_logical(v, jnp.int32(sh)) & jnp.int32(0xFF)

    def hist_phase(p, r, kIN, iIN):
        sh = 8 * p
        if p == 0:
            pltpu.sync_copy(x_hbm.at[r, pl.ds(base, CHUNK)], rawf)

            def tl(j, _):
                b = plsc.bitcast(rawf[pl.ds(j * L, L)], jnp.int32)
                m = lax.shift_right_arithmetic(b, 31)
                kd = (b ^ (m | jnp.int32(-2147483648))) ^ jnp.int32(-1)
                kch[pl.ds(j * L, L)] = kd
                ich[pl.ds(j * L, L)] = base + j * L + iota
                return 0

            lax.fori_loop(0, NV, tl, 0, unroll=4)
        else:
            pltpu.sync_copy(kIN.at[pl.ds(base, CHUNK)], kch)
            pltpu.sync_copy(iIN.at[pl.ds(base, CHUNK)], ich)

        def z(j, _):
            hloc[pl.ds(j * L, L)] = zero16
            return 0

        lax.fori_loop(0, RADIX // L, z, 0)

        def hb(j, _):
            d = digit(kch[pl.ds(j * L, L)], sh)
            occ, lastm = plsc.scan_count(d)
            plsc.addupdate_scatter(hloc, [d], occ, mask=lastm)
            return 0

        lax.fori_loop(0, NV, hb, 0, unroll=4)
        pltpu.sync_copy(hloc, hgrid.at[t])
        plsc.subcore_barrier()

    def scan_phase():
        pltpu.sync_copy(hgrid, gloc)

        def g_body(g, runv):
            part = zero16
            tot = zero16
            for tp in range(NT):
                rowv = gloc[tp, pl.ds(g * L, L)]
                part = part + jnp.where(tp < t, rowv, zero16)
                tot = tot + rowv
            csum = plsc.cumsum(tot)
            offs[pl.ds(g * L, L)] = runv + (csum - tot) + part
            return runv + jnp.full((L,), jnp.sum(tot), jnp.int32)

        lax.fori_loop(0, RADIX // L, g_body, zero16)

    def permute_phase(p, kOUT, iOUT):
        sh = 8 * p

        def pb(j, _):
            v = kch[pl.ds(j * L, L)]
            d = digit(v, sh)
            occ, lastm = plsc.scan_count(d)
            bse = plsc.load_gather(offs, [d])
            plsc.store_scatter(offs, [d], bse + occ, mask=lastm)
            posg[j // 8, pl.ds((j % 8) * L, L)] = bse + occ - 1
            return 0

        lax.fori_loop(0, NV, pb, 0, unroll=2)
        handles = []
        for w in range(NW):
            if kOUT is not None:
                handles.append(pltpu.async_copy(
                    kch.at[pl.ds(w * 128, 128)], kOUT.at[posg.at[w]], sem))
            handles.append(pltpu.async_copy(
                ich.at[pl.ds(w * 128, 128)], iOUT.at[posg.at[w]], sem))
        for h in handles:
            h.wait()
        plsc.subcore_barrier()

    def row_body(rr, _):
        r = rr * NC + c
        hist_phase(0, r, None, None)
        scan_phase()
        permute_phase(0, kA, iA)
        hist_phase(1, r, kA, iA)
        scan_phase()
        permute_phase(1, kB, iB)
        hist_phase(2, r, kB, iB)
        scan_phase()
        permute_phase(2, kA, iA)
        hist_phase(3, r, kA, iA)
        scan_phase()
        permute_phase(3, None, iB)
        pltpu.sync_copy(iB.at[pl.ds(base, CHUNK)],
                        out_hbm.at[r, pl.ds(base, CHUNK)])
        plsc.subcore_barrier()
        return 0

    lax.fori_loop(0, ROWS_PER_SC, row_body, 0)


def kernel(sort_ip):
    xp = jnp.pad(sort_ip, ((0, 0), (0, NP - N0)),
                 constant_values=-jnp.inf)
    mesh = plsc.VectorSubcoreMesh(core_axis_name="c", subcore_axis_name="s")
    fn = pl.kernel(
        _sort_body,
        out_type=jax.ShapeDtypeStruct((NR, NP), jnp.int32),
        mesh=mesh,
        compiler_params=pltpu.CompilerParams(needs_layout_passes=False),
        scratch_types=(
            pltpu.VMEM_SHARED((NP,), jnp.int32),
            pltpu.VMEM_SHARED((NP,), jnp.int32),
            pltpu.VMEM_SHARED((NP,), jnp.int32),
            pltpu.VMEM_SHARED((NP,), jnp.int32),
            pltpu.VMEM_SHARED((NT, RADIX), jnp.int32),
            pltpu.VMEM((CHUNK,), jnp.float32),
            pltpu.VMEM((CHUNK,), jnp.int32),
            pltpu.VMEM((CHUNK,), jnp.int32),
            pltpu.VMEM((NW, 128), jnp.int32),
            pltpu.VMEM((RADIX,), jnp.int32),
            pltpu.VMEM((RADIX,), jnp.int32),
            pltpu.VMEM((NT, RADIX), jnp.int32),
            pltpu.SemaphoreType.DMA,
        ),
    )
    out = fn(xp)
    return out[:, :N0]


# speedup vs baseline: 2.7536x; 1.0102x over previous
import jax
import jax.numpy as jnp
from jax import lax
from jax.experimental import pallas as pl
from jax.experimental.pallas import tpu as pltpu
from jax.experimental.pallas import tpu_sc as plsc

NR = 64
N0 = 100000
L = 16
NT = 16
NC = 2
CHUNK = 6272
NP = NT * CHUNK
NW = CHUNK // 128
NV = CHUNK // L
RADIX = 256
PAIRS_PER_SC = NR // NC // 2


def _sort_body(x_hbm, out_hbm, kPA, iPA, kPB, iPB, hgrid,
               rawf, kch, ich, posg, offs, hloc, gloc, sem):
    c = lax.axis_index("c")
    t = lax.axis_index("s")
    iota = lax.iota(jnp.int32, L)
    base = t * CHUNK
    zero16 = jnp.zeros((L,), jnp.int32)

    def digit(v, sh, u):
        d = lax.shift_right_logical(v, jnp.int32(sh)) & jnp.int32(0xFF)
        return d + jnp.int32(u * RADIX)

    def hist_phase(p, rows, kIN, iIN):
        sh = 8 * p
        if p == 0:
            for u, r in enumerate(rows):
                pltpu.sync_copy(x_hbm.at[r, pl.ds(base, CHUNK)],
                                rawf.at[pl.ds(u * CHUNK, CHUNK)])

            def tl(j, _):
                for u in range(2):
                    o = u * CHUNK + j * L
                    b = plsc.bitcast(rawf[pl.ds(o, L)], jnp.int32)
                    m = lax.shift_right_arithmetic(b, 31)
                    kd = (b ^ (m | jnp.int32(-2147483648))) ^ jnp.int32(-1)
                    kch[pl.ds(o, L)] = kd
                    ich[pl.ds(o, L)] = base + j * L + iota
                return 0

            lax.fori_loop(0, NV, tl, 0, unroll=2)
        else:
            hs = []
            for u in range(2):
                hs.append(pltpu.async_copy(
                    kIN.at[pl.ds(u * NP + base, CHUNK)],
                    kch.at[pl.ds(u * CHUNK, CHUNK)], sem))
                hs.append(pltpu.async_copy(
                    iIN.at[pl.ds(u * NP + base, CHUNK)],
                    ich.at[pl.ds(u * CHUNK, CHUNK)], sem))
            for h in hs:
                h.wait()

        def z(j, _):
            hloc[pl.ds(j * L, L)] = zero16
            return 0

        lax.fori_loop(0, 2 * RADIX // L, z, 0)

        def hb(j, _):
            for u in range(2):
                d = digit(kch[pl.ds(u * CHUNK + j * L, L)], sh, u)
                occ, lastm = plsc.scan_count(d)
                plsc.addupdate_scatter(hloc, [d], occ, mask=lastm)
            return 0

        lax.fori_loop(0, NV, hb, 0, unroll=2)
        pltpu.sync_copy(hloc.at[pl.ds(0, RADIX)], hgrid.at[t])
        pltpu.sync_copy(hloc.at[pl.ds(RADIX, RADIX)], hgrid.at[NT + t])
        plsc.subcore_barrier()

    def scan_phase():
        for u in range(2):
            pltpu.sync_copy(hgrid.at[pl.ds(u * NT, NT)], gloc)

            def g_body(g, runv):
                part = zero16
                tot = zero16
                for tp in range(NT):
                    rowv = gloc[tp, pl.ds(g * L, L)]
                    part = part + jnp.where(tp < t, rowv, zero16)
                    tot = tot + rowv
                csum = plsc.cumsum(tot)
                offs[pl.ds(u * RADIX + g * L, L)] = (
                    runv + (csum - tot) + part)
                return runv + jnp.full((L,), jnp.sum(tot), jnp.int32)

            lax.fori_loop(0, RADIX // L, g_body, zero16)

    def permute_phase(p, kOUT, iOUT):
        sh = 8 * p

        def pb(j, _):
            for u in range(2):
                v = kch[pl.ds(u * CHUNK + j * L, L)]
                d = digit(v, sh, u)
                occ, lastm = plsc.scan_count(d)
                bse = plsc.load_gather(offs, [d])
                plsc.store_scatter(offs, [d], bse + occ, mask=lastm)
                posg[u * NW + j // 8, pl.ds((j % 8) * L, L)] = (
                    bse + occ - 1 + u * NP)
            return 0

        lax.fori_loop(0, NV, pb, 0, unroll=2)
        handles = []
        for u in range(2):
            for w in range(NW):
                if kOUT is not None:
                    handles.append(pltpu.async_copy(
                        kch.at[pl.ds(u * CHUNK + w * 128, 128)],
                        kOUT.at[posg.at[u * NW + w]], sem))
                handles.append(pltpu.async_copy(
                    ich.at[pl.ds(u * CHUNK + w * 128, 128)],
                    iOUT.at[posg.at[u * NW + w]], sem))
        for h in handles:
            h.wait()
        plsc.subcore_barrier()

    def pair_body(q, _):
        rows = (2 * q * NC + c, (2 * q + 1) * NC + c)
        hist_phase(0, rows, None, None)
        scan_phase()
        permute_phase(0, kPA, iPA)
        hist_phase(1, rows, kPA, iPA)
        scan_phase()
        permute_phase(1, kPB, iPB)
        hist_phase(2, rows, kPB, iPB)
        scan_phase()
        permute_phase(2, kPA, iPA)
        hist_phase(3, rows, kPA, iPA)
        scan_phase()
        permute_phase(3, None, iPB)
        for u, r in enumerate(rows):
            pltpu.sync_copy(iPB.at[pl.ds(u * NP + base, CHUNK)],
                            out_hbm.at[r, pl.ds(base, CHUNK)])
        plsc.subcore_barrier()
        return 0

    lax.fori_loop(0, PAIRS_PER_SC, pair_body, 0)


def kernel(sort_ip):
    xp = jnp.pad(sort_ip, ((0, 0), (0, NP - N0)),
                 constant_values=-jnp.inf)
    mesh = plsc.VectorSubcoreMesh(core_axis_name="c", subcore_axis_name="s")
    fn = pl.kernel(
        _sort_body,
        out_type=jax.ShapeDtypeStruct((NR, NP), jnp.int32),
        mesh=mesh,
        compiler_params=pltpu.CompilerParams(needs_layout_passes=False),
        scratch_types=(
            pltpu.VMEM_SHARED((2 * NP,), jnp.int32),
            pltpu.VMEM_SHARED((2 * NP,), jnp.int32),
            pltpu.VMEM_SHARED((2 * NP,), jnp.int32),
            pltpu.VMEM_SHARED((2 * NP,), jnp.int32),
            pltpu.VMEM_SHARED((2 * NT, RADIX), jnp.int32),
            pltpu.VMEM((2 * CHUNK,), jnp.float32),
            pltpu.VMEM((2 * CHUNK,), jnp.int32),
            pltpu.VMEM((2 * CHUNK,), jnp.int32),
            pltpu.VMEM((2 * NW, 128), jnp.int32),
            pltpu.VMEM((2 * RADIX,), jnp.int32),
            pltpu.VMEM((2 * RADIX,), jnp.int32),
            pltpu.VMEM((NT, RADIX), jnp.int32),
            pltpu.SemaphoreType.DMA,
        ),
    )
    out = fn(xp)
    return out[:, :N0]


# speedup vs baseline: 2.7863x; 1.0119x over previous
import jax
import jax.numpy as jnp
from jax import lax
from jax.experimental import pallas as pl
from jax.experimental.pallas import tpu as pltpu
from jax.experimental.pallas import tpu_sc as plsc

NR = 64
N0 = 100000
L = 16
NT = 16
NC = 2
CHUNK = 6272
NP = NT * CHUNK
NW = CHUNK // 128
NV = CHUNK // L
RADIX = 256
PAIRS_PER_SC = NR // NC // 2


def _sort_body(x_hbm, out_hbm, kPA, iPA, kPB, iPB, hgrid,
               rawf0, rawf1, kch0, kch1, ich0, ich1, posg0, posg1,
               offs0, offs1, hloc0, hloc1, gloc, sem):
    rawf = (rawf0, rawf1)
    kch = (kch0, kch1)
    ich = (ich0, ich1)
    posg = (posg0, posg1)
    offs = (offs0, offs1)
    hloc = (hloc0, hloc1)
    c = lax.axis_index("c")
    t = lax.axis_index("s")
    iota = lax.iota(jnp.int32, L)
    base = t * CHUNK
    zero16 = jnp.zeros((L,), jnp.int32)

    def digit(v, sh):
        return lax.shift_right_logical(v, jnp.int32(sh)) & jnp.int32(0xFF)

    def hist_phase(p, rows, kIN, iIN):
        sh = 8 * p
        if p == 0:
            for u, r in enumerate(rows):
                pltpu.sync_copy(x_hbm.at[r, pl.ds(base, CHUNK)], rawf[u])

            def tl(j, _):
                for u in range(2):
                    b = plsc.bitcast(rawf[u][pl.ds(j * L, L)], jnp.int32)
                    m = lax.shift_right_arithmetic(b, 31)
                    kd = (b ^ (m | jnp.int32(-2147483648))) ^ jnp.int32(-1)
                    kch[u][pl.ds(j * L, L)] = kd
                    ich[u][pl.ds(j * L, L)] = base + j * L + iota
                return 0

            lax.fori_loop(0, NV, tl, 0, unroll=2)
        else:
            hs = []
            for u in range(2):
                hs.append(pltpu.async_copy(
                    kIN.at[pl.ds(u * NP + base, CHUNK)], kch[u], sem))
                hs.append(pltpu.async_copy(
                    iIN.at[pl.ds(u * NP + base, CHUNK)], ich[u], sem))
            for h in hs:
                h.wait()

        def z(j, _):
            hloc[0][pl.ds(j * L, L)] = zero16
            hloc[1][pl.ds(j * L, L)] = zero16
            return 0

        lax.fori_loop(0, RADIX // L, z, 0)

        def hb(j, _):
            for u in range(2):
                d = digit(kch[u][pl.ds(j * L, L)], sh)
                occ, lastm = plsc.scan_count(d)
                plsc.addupdate_scatter(hloc[u], [d], occ, mask=lastm)
            return 0

        lax.fori_loop(0, NV, hb, 0, unroll=2)
        pltpu.sync_copy(hloc[0], hgrid.at[t])
        pltpu.sync_copy(hloc[1], hgrid.at[NT + t])
        plsc.subcore_barrier()

    def scan_phase():
        for u in range(2):
            pltpu.sync_copy(hgrid.at[pl.ds(u * NT, NT)], gloc)

            def g_body(g, runv):
                part = zero16
                tot = zero16
                for tp in range(NT):
                    rowv = gloc[tp, pl.ds(g * L, L)]
                    part = part + jnp.where(tp < t, rowv, zero16)
                    tot = tot + rowv
                csum = plsc.cumsum(tot)
                offs[u][pl.ds(g * L, L)] = runv + (csum - tot) + part
                return runv + jnp.full((L,), jnp.sum(tot), jnp.int32)

            lax.fori_loop(0, RADIX // L, g_body, zero16)

    def permute_phase(p, kOUT, iOUT):
        sh = 8 * p

        def pb(j, _):
            for u in range(2):
                v = kch[u][pl.ds(j * L, L)]
                d = digit(v, sh)
                occ, lastm = plsc.scan_count(d)
                bse = plsc.load_gather(offs[u], [d])
                plsc.store_scatter(offs[u], [d], bse + occ, mask=lastm)
                posg[u][j // 8, pl.ds((j % 8) * L, L)] = (
                    bse + occ - 1 + u * NP)
            return 0

        lax.fori_loop(0, NV, pb, 0, unroll=2)
        handles = []
        for u in range(2):
            for w in range(NW):
                if kOUT is not None:
                    handles.append(pltpu.async_copy(
                        kch[u].at[pl.ds(w * 128, 128)],
                        kOUT.at[posg[u].at[w]], sem))
                handles.append(pltpu.async_copy(
                    ich[u].at[pl.ds(w * 128, 128)],
                    iOUT.at[posg[u].at[w]], sem))
        for h in handles:
            h.wait()
        plsc.subcore_barrier()

    def pair_body(q, _):
        rows = (2 * q * NC + c, (2 * q + 1) * NC + c)
        hist_phase(0, rows, None, None)
        scan_phase()
        permute_phase(0, kPA, iPA)
        hist_phase(1, rows, kPA, iPA)
        scan_phase()
        permute_phase(1, kPB, iPB)
        hist_phase(2, rows, kPB, iPB)
        scan_phase()
        permute_phase(2, kPA, iPA)
        hist_phase(3, rows, kPA, iPA)
        scan_phase()
        permute_phase(3, None, iPB)
        for u, r in enumerate(rows):
            pltpu.sync_copy(iPB.at[pl.ds(u * NP + base, CHUNK)],
                            out_hbm.at[r, pl.ds(base, CHUNK)])
        plsc.subcore_barrier()
        return 0

    lax.fori_loop(0, PAIRS_PER_SC, pair_body, 0)


def kernel(sort_ip):
    xp = jnp.pad(sort_ip, ((0, 0), (0, NP - N0)),
                 constant_values=-jnp.inf)
    mesh = plsc.VectorSubcoreMesh(core_axis_name="c", subcore_axis_name="s")
    fn = pl.kernel(
        _sort_body,
        out_type=jax.ShapeDtypeStruct((NR, NP), jnp.int32),
        mesh=mesh,
        compiler_params=pltpu.CompilerParams(needs_layout_passes=False),
        scratch_types=(
            pltpu.VMEM_SHARED((2 * NP,), jnp.int32),
            pltpu.VMEM_SHARED((2 * NP,), jnp.int32),
            pltpu.VMEM_SHARED((2 * NP,), jnp.int32),
            pltpu.VMEM_SHARED((2 * NP,), jnp.int32),
            pltpu.VMEM_SHARED((2 * NT, RADIX), jnp.int32),
            pltpu.VMEM((CHUNK,), jnp.float32),
            pltpu.VMEM((CHUNK,), jnp.float32),
            pltpu.VMEM((CHUNK,), jnp.int32),
            pltpu.VMEM((CHUNK,), jnp.int32),
            pltpu.VMEM((CHUNK,), jnp.int32),
            pltpu.VMEM((CHUNK,), jnp.int32),
            pltpu.VMEM((NW, 128), jnp.int32),
            pltpu.VMEM((NW, 128), jnp.int32),
            pltpu.VMEM((RADIX,), jnp.int32),
            pltpu.VMEM((RADIX,), jnp.int32),
            pltpu.VMEM((RADIX,), jnp.int32),
            pltpu.VMEM((RADIX,), jnp.int32),
            pltpu.VMEM((NT, RADIX), jnp.int32),
            pltpu.SemaphoreType.DMA,
        ),
    )
    out = fn(xp)
    return out[:, :N0]


# speedup vs baseline: 2.8276x; 1.0148x over previous
import jax
import jax.numpy as jnp
from jax import lax
from jax.experimental import pallas as pl
from jax.experimental.pallas import tpu as pltpu
from jax.experimental.pallas import tpu_sc as plsc

NR = 64
N0 = 100000
L = 16
NT = 16
NC = 2
CHUNK = 6272
NP = NT * CHUNK
NW = CHUNK // 128
NV = CHUNK // L
R0, R1, R2, R3 = 512, 256, 256, 128
PAIRS_PER_SC = NR // NC // 2


def _sort_body(x_hbm, out_hbm, bufK, bufI, bufP, hg512, hg256, hg128,
               rawf0, rawf1, kch0, kch1, ich0, ich1, pch0, pch1,
               posg0, posg1, offs0, offs1, hloc0, hloc1,
               gl512, gl256, gl128, sem):
    rawf = (rawf0, rawf1)
    kch = (kch0, kch1)
    ich = (ich0, ich1)
    pch = (pch0, pch1)
    posg = (posg0, posg1)
    offs = (offs0, offs1)
    hloc = (hloc0, hloc1)
    c = lax.axis_index("c")
    t = lax.axis_index("s")
    iota = lax.iota(jnp.int32, L)
    base = t * CHUNK
    zero16 = jnp.zeros((L,), jnp.int32)

    digit_of = (
        lambda v: v & jnp.int32(0x1FF),
        lambda v: lax.shift_right_logical(v, jnp.int32(9)) & jnp.int32(0xFF),
        lambda v: lax.shift_right_logical(v, jnp.int32(17)) & jnp.int32(0xFF),
        lambda v: lax.shift_right_logical(v, jnp.int32(25)) & jnp.int32(0x7F),
    )
    payload_of = (
        None,
        lambda kv, iv: (kv & jnp.int32(-131072)) | iv,
        lambda kv, iv: kv & jnp.int32(-33423361),
        lambda kv, iv: kv & jnp.int32(0x1FFFF),
    )

    def load_phase(p, rows, srcK, srcI):
        if p == 0:
            for u, r in enumerate(rows):
                pltpu.sync_copy(x_hbm.at[r, pl.ds(base, CHUNK)], rawf[u])

            def tl(j, _):
                for u in range(2):
                    b = plsc.bitcast(rawf[u][pl.ds(j * L, L)], jnp.int32)
                    m = lax.shift_right_arithmetic(b, 31)
                    kd = (b ^ (m | jnp.int32(-2147483648))) ^ jnp.int32(-1)
                    kch[u][pl.ds(j * L, L)] = kd
                    ich[u][pl.ds(j * L, L)] = base + j * L + iota
                return 0

            lax.fori_loop(0, NV, tl, 0, unroll=2)
        else:
            hs = []
            for u in range(2):
                hs.append(pltpu.async_copy(
                    srcK.at[pl.ds(u * NP + base, CHUNK)], kch[u], sem))
                if srcI is not None:
                    hs.append(pltpu.async_copy(
                        srcI.at[pl.ds(u * NP + base, CHUNK)], ich[u], sem))
            for h in hs:
                h.wait()

    def hist_phase(p, radix, hgrid):
        dig = digit_of[p]

        def z(j, _):
            hloc[0][pl.ds(j * L, L)] = zero16
            hloc[1][pl.ds(j * L, L)] = zero16
            return 0

        lax.fori_loop(0, radix // L, z, 0)

        def hb(j, _):
            for u in range(2):
                d = dig(kch[u][pl.ds(j * L, L)])
                occ, lastm = plsc.scan_count(d)
                plsc.addupdate_scatter(hloc[u], [d], occ, mask=lastm)
            return 0

        lax.fori_loop(0, NV, hb, 0, unroll=2)
        pltpu.sync_copy(hloc[0].at[pl.ds(0, radix)], hgrid.at[t])
        pltpu.sync_copy(hloc[1].at[pl.ds(0, radix)], hgrid.at[NT + t])
        plsc.subcore_barrier()

    def scan_phase(radix, hgrid, gloc):
        for u in range(2):
            pltpu.sync_copy(hgrid.at[pl.ds(u * NT, NT)], gloc)

            def g_body(g, runv):
                part = zero16
                tot = zero16
                for tp in range(NT):
                    rowv = gloc[tp, pl.ds(g * L, L)]
                    part = part + jnp.where(tp < t, rowv, zero16)
                    tot = tot + rowv
                csum = plsc.cumsum(tot)
                offs[u][pl.ds(g * L, L)] = runv + (csum - tot) + part
                return runv + jnp.full((L,), jnp.sum(tot), jnp.int32)

            lax.fori_loop(0, radix // L, g_body, zero16)

    def permute_phase(p, dstK, dstI):
        dig = digit_of[p]
        pld = payload_of[p]

        def pb(j, _):
            for u in range(2):
                kv = kch[u][pl.ds(j * L, L)]
                d = dig(kv)
                occ, lastm = plsc.scan_count(d)
                bse = plsc.load_gather(offs[u], [d])
                plsc.store_scatter(offs[u], [d], bse + occ, mask=lastm)
                posg[u][j // 8, pl.ds((j % 8) * L, L)] = (
                    bse + occ - 1 + u * NP)
                if pld is not None:
                    iv = ich[u][pl.ds(j * L, L)] if p == 1 else None
                    pch[u][pl.ds(j * L, L)] = pld(kv, iv)
            return 0

        lax.fori_loop(0, NV, pb, 0, unroll=2)
        handles = []
        for u in range(2):
            for w in range(NW):
                if p == 0:
                    handles.append(pltpu.async_copy(
                        kch[u].at[pl.ds(w * 128, 128)],
                        dstK.at[posg[u].at[w]], sem))
                    handles.append(pltpu.async_copy(
                        ich[u].at[pl.ds(w * 128, 128)],
                        dstI.at[posg[u].at[w]], sem))
                else:
                    handles.append(pltpu.async_copy(
                        pch[u].at[pl.ds(w * 128, 128)],
                        dstK.at[posg[u].at[w]], sem))
        for h in handles:
            h.wait()
        plsc.subcore_barrier()

    def pair_body(q, _):
        rows = (2 * q * NC + c, (2 * q + 1) * NC + c)
        load_phase(0, rows, None, None)
        hist_phase(0, R0, hg512)
        scan_phase(R0, hg512, gl512)
        permute_phase(0, bufK, bufI)
        load_phase(1, rows, bufK, bufI)
        hist_phase(1, R1, hg256)
        scan_phase(R1, hg256, gl256)
        permute_phase(1, bufP, None)
        load_phase(2, rows, bufP, None)
        hist_phase(2, R2, hg256)
        scan_phase(R2, hg256, gl256)
        permute_phase(2, bufK, None)
        load_phase(3, rows, bufK, None)
        hist_phase(3, R3, hg128)
        scan_phase(R3, hg128, gl128)
        permute_phase(3, bufP, None)
        for u, r in enumerate(rows):
            pltpu.sync_copy(bufP.at[pl.ds(u * NP + base, CHUNK)],
                            out_hbm.at[r, pl.ds(base, CHUNK)])
        plsc.subcore_barrier()
        return 0

    lax.fori_loop(0, PAIRS_PER_SC, pair_body, 0)


def kernel(sort_ip):
    xp = jnp.pad(sort_ip, ((0, 0), (0, NP - N0)),
                 constant_values=-jnp.inf)
    mesh = plsc.VectorSubcoreMesh(core_axis_name="c", subcore_axis_name="s")
    fn = pl.kernel(
        _sort_body,
        out_type=jax.ShapeDtypeStruct((NR, NP), jnp.int32),
        mesh=mesh,
        compiler_params=pltpu.CompilerParams(needs_layout_passes=False),
        scratch_types=(
            pltpu.VMEM_SHARED((2 * NP,), jnp.int32),
            pltpu.VMEM_SHARED((2 * NP,), jnp.int32),
            pltpu.VMEM_SHARED((2 * NP,), jnp.int32),
            pltpu.VMEM_SHARED((2 * NT, R0), jnp.int32),
            pltpu.VMEM_SHARED((2 * NT, R1), jnp.int32),
            pltpu.VMEM_SHARED((2 * NT, R3), jnp.int32),
            pltpu.VMEM((CHUNK,), jnp.float32),
            pltpu.VMEM((CHUNK,), jnp.float32),
            pltpu.VMEM((CHUNK,), jnp.int32),
            pltpu.VMEM((CHUNK,), jnp.int32),
            pltpu.VMEM((CHUNK,), jnp.int32),
            pltpu.VMEM((CHUNK,), jnp.int32),
            pltpu.VMEM((CHUNK,), jnp.int32),
            pltpu.VMEM((CHUNK,), jnp.int32),
            pltpu.VMEM((NW, 128), jnp.int32),
            pltpu.VMEM((NW, 128), jnp.int32),
            pltpu.VMEM((R0,), jnp.int32),
            pltpu.VMEM((R0,), jnp.int32),
            pltpu.VMEM((R0,), jnp.int32),
            pltpu.VMEM((R0,), jnp.int32),
            pltpu.VMEM((NT, R0), jnp.int32),
            pltpu.VMEM((NT, R1), jnp.int32),
            pltpu.VMEM((NT, R3), jnp.int32),
            pltpu.SemaphoreType.DMA,
        ),
    )
    out = fn(xp)
    return out[:, :N0]


# speedup vs baseline: 3.5012x; 1.2382x over previous
import jax
import jax.numpy as jnp
from jax import lax
from jax.experimental import pallas as pl
from jax.experimental.pallas import tpu as pltpu
from jax.experimental.pallas import tpu_sc as plsc

NR = 64
N0 = 100000
L = 16
NT = 16
NC = 2
CHUNK = 6272
NP = NT * CHUNK
NW = CHUNK // 128
NV = CHUNK // L
R0, R1, R2, R3 = 512, 256, 256, 128
PAIRS_PER_SC = NR // NC // 2


def _sort_body(x_hbm, out_hbm, bufK, bufI, bufP, hg512, hg256, hg128,
               rawf0, rawf1, kch0, kch1, ich0, ich1, pch0, pch1,
               posg0, posg1, offs0, offs1, hloc0, hloc1,
               gl512, gl256, gl128, sem):
    rawf = (rawf0, rawf1)
    kch = (kch0, kch1)
    ich = (ich0, ich1)
    pch = (pch0, pch1)
    posg = (posg0, posg1)
    offs = (offs0, offs1)
    hloc = (hloc0, hloc1)
    c = lax.axis_index("c")
    t = lax.axis_index("s")
    iota = lax.iota(jnp.int32, L)
    base = t * CHUNK
    zero16 = jnp.zeros((L,), jnp.int32)

    digit_of = (
        lambda v: v & jnp.int32(0x1FF),
        lambda v: lax.shift_right_logical(v, jnp.int32(9)) & jnp.int32(0xFF),
        lambda v: lax.shift_right_logical(v, jnp.int32(17)) & jnp.int32(0xFF),
        lambda v: lax.shift_right_logical(v, jnp.int32(25)) & jnp.int32(0x7F),
    )
    payload_of = (
        None,
        lambda kv, iv: (kv & jnp.int32(-131072)) | iv,
        lambda kv, iv: kv & jnp.int32(-33423361),
        lambda kv, iv: kv & jnp.int32(0x1FFFF),
    )

    def load_phase(p, rows, srcK, srcI):
        if p == 0:
            for u, r in enumerate(rows):
                pltpu.sync_copy(x_hbm.at[r, pl.ds(base, CHUNK)], rawf[u])

            def tl(j, _):
                for u in range(2):
                    b = plsc.bitcast(rawf[u][pl.ds(j * L, L)], jnp.int32)
                    m = lax.shift_right_arithmetic(b, 31)
                    kd = (b ^ (m | jnp.int32(-2147483648))) ^ jnp.int32(-1)
                    kch[u][pl.ds(j * L, L)] = kd
                    ich[u][pl.ds(j * L, L)] = base + j * L + iota
                return 0

            lax.fori_loop(0, NV, tl, 0, unroll=2)
        else:
            hs = []
            for u in range(2):
                hs.append(pltpu.async_copy(
                    srcK.at[pl.ds(u * NP + base, CHUNK)], kch[u], sem))
                if srcI is not None:
                    hs.append(pltpu.async_copy(
                        srcI.at[pl.ds(u * NP + base, CHUNK)], ich[u], sem))
            for h in hs:
                h.wait()

    def hist_phase(p, radix, hgrid):
        dig = digit_of[p]

        def z(j, _):
            hloc[0][pl.ds(j * L, L)] = zero16
            hloc[1][pl.ds(j * L, L)] = zero16
            return 0

        lax.fori_loop(0, radix // L, z, 0)

        def hb(j, _):
            for u in range(2):
                d = dig(kch[u][pl.ds(j * L, L)])
                occ, lastm = plsc.scan_count(d)
                plsc.addupdate_scatter(hloc[u], [d], occ, mask=lastm)
            return 0

        lax.fori_loop(0, NV, hb, 0, unroll=2)
        pltpu.sync_copy(hloc[0].at[pl.ds(0, radix)], hgrid.at[t])
        pltpu.sync_copy(hloc[1].at[pl.ds(0, radix)], hgrid.at[NT + t])
        plsc.subcore_barrier()

    def scan_phase(radix, hgrid, gloc):
        for u in range(2):
            pltpu.sync_copy(hgrid.at[pl.ds(u * NT, NT)], gloc)

            def g_body(g, runv):
                part = zero16
                tot = zero16
                for tp in range(NT):
                    rowv = gloc[tp, pl.ds(g * L, L)]
                    part = part + jnp.where(tp < t, rowv, zero16)
                    tot = tot + rowv
                csum = plsc.cumsum(tot)
                offs[u][pl.ds(g * L, L)] = runv + (csum - tot) + part
                return runv + jnp.full((L,), jnp.sum(tot), jnp.int32)

            lax.fori_loop(0, radix // L, g_body, zero16)

    def permute_phase(p, dstK, dstI):
        dig = digit_of[p]
        pld = payload_of[p]

        def sc_at(u, j):
            d = dig(kch[u][pl.ds(j * L, L)])
            occ, lastm = plsc.scan_count(d)
            return d, occ, lastm

        dA, oA, lA = sc_at(0, 0)
        dB, oB, lB = sc_at(1, 0)

        def pb(j, carry):
            jn = jnp.minimum(j + 1, NV - 1)
            nxt = sc_at(0, jn) + sc_at(1, jn)
            cur = (carry[0:3], carry[3:6])
            for u in range(2):
                d, occ, lastm = cur[u]
                bse = plsc.load_gather(offs[u], [d])
                plsc.store_scatter(offs[u], [d], bse + occ, mask=lastm)
                posg[u][j // 8, pl.ds((j % 8) * L, L)] = (
                    bse + occ - 1 + u * NP)
                if pld is not None:
                    kv = kch[u][pl.ds(j * L, L)]
                    iv = ich[u][pl.ds(j * L, L)] if p == 1 else None
                    pch[u][pl.ds(j * L, L)] = pld(kv, iv)
            return nxt

        lax.fori_loop(0, NV, pb, (dA, oA, lA, dB, oB, lB), unroll=2)
        handles = []
        for u in range(2):
            for w in range(NW):
                if p == 0:
                    handles.append(pltpu.async_copy(
                        kch[u].at[pl.ds(w * 128, 128)],
                        dstK.at[posg[u].at[w]], sem))
                    handles.append(pltpu.async_copy(
                        ich[u].at[pl.ds(w * 128, 128)],
                        dstI.at[posg[u].at[w]], sem))
                else:
                    handles.append(pltpu.async_copy(
                        pch[u].at[pl.ds(w * 128, 128)],
                        dstK.at[posg[u].at[w]], sem))
        for h in handles:
            h.wait()
        plsc.subcore_barrier()

    def pair_body(q, _):
        rows = (2 * q * NC + c, (2 * q + 1) * NC + c)
        load_phase(0, rows, None, None)
        hist_phase(0, R0, hg512)
        scan_phase(R0, hg512, gl512)
        permute_phase(0, bufK, bufI)
        load_phase(1, rows, bufK, bufI)
        hist_phase(1, R1, hg256)
        scan_phase(R1, hg256, gl256)
        permute_phase(1, bufP, None)
        load_phase(2, rows, bufP, None)
        hist_phase(2, R2, hg256)
        scan_phase(R2, hg256, gl256)
        permute_phase(2, bufK, None)
        load_phase(3, rows, bufK, None)
        hist_phase(3, R3, hg128)
        scan_phase(R3, hg128, gl128)
        permute_phase(3, bufP, None)
        for u, r in enumerate(rows):
            pltpu.sync_copy(bufP.at[pl.ds(u * NP + base, CHUNK)],
                            out_hbm.at[r, pl.ds(base, CHUNK)])
        plsc.subcore_barrier()
        return 0

    lax.fori_loop(0, PAIRS_PER_SC, pair_body, 0)


def kernel(sort_ip):
    xp = jnp.pad(sort_ip, ((0, 0), (0, NP - N0)),
                 constant_values=-jnp.inf)
    mesh = plsc.VectorSubcoreMesh(core_axis_name="c", subcore_axis_name="s")
    fn = pl.kernel(
        _sort_body,
        out_type=jax.ShapeDtypeStruct((NR, NP), jnp.int32),
        mesh=mesh,
        compiler_params=pltpu.CompilerParams(needs_layout_passes=False),
        scratch_types=(
            pltpu.VMEM_SHARED((2 * NP,), jnp.int32),
            pltpu.VMEM_SHARED((2 * NP,), jnp.int32),
            pltpu.VMEM_SHARED((2 * NP,), jnp.int32),
            pltpu.VMEM_SHARED((2 * NT, R0), jnp.int32),
            pltpu.VMEM_SHARED((2 * NT, R1), jnp.int32),
            pltpu.VMEM_SHARED((2 * NT, R3), jnp.int32),
            pltpu.VMEM((CHUNK,), jnp.float32),
            pltpu.VMEM((CHUNK,), jnp.float32),
            pltpu.VMEM((CHUNK,), jnp.int32),
            pltpu.VMEM((CHUNK,), jnp.int32),
            pltpu.VMEM((CHUNK,), jnp.int32),
            pltpu.VMEM((CHUNK,), jnp.int32),
            pltpu.VMEM((CHUNK,), jnp.int32),
            pltpu.VMEM((CHUNK,), jnp.int32),
            pltpu.VMEM((NW, 128), jnp.int32),
            pltpu.VMEM((NW, 128), jnp.int32),
            pltpu.VMEM((R0,), jnp.int32),
            pltpu.VMEM((R0,), jnp.int32),
            pltpu.VMEM((R0,), jnp.int32),
            pltpu.VMEM((R0,), jnp.int32),
            pltpu.VMEM((NT, R0), jnp.int32),
            pltpu.VMEM((NT, R1), jnp.int32),
            pltpu.VMEM((NT, R3), jnp.int32),
            pltpu.SemaphoreType.DMA,
        ),
    )
    out = fn(xp)
    return out[:, :N0]


# speedup vs baseline: 4.9901x; 1.4253x over previous
import jax
import jax.numpy as jnp
from jax import lax
from jax.experimental import pallas as pl
from jax.experimental.pallas import tpu as pltpu
from jax.experimental.pallas import tpu_sc as plsc

NR = 64
N0 = 100000
L = 16
NT = 16
NC = 2
CHUNK = 6272
NP = NT * CHUNK
NW = CHUNK // 128
NV = CHUNK // L
R0, R1, R2, R3 = 512, 256, 256, 128
PAIRS_PER_SC = NR // NC // 2


def _sort_body(x_hbm, out_hbm, bufK, bufI, bufP, hg512, hg256, hg128,
               rawf0, rawf1, kch0, kch1, ich0, ich1, pch0, pch1,
               posg0, posg1, offs0, offs1, hloc0, hloc1,
               gl512, gl256, gl128, sem):
    rawf = (rawf0, rawf1)
    kch = (kch0, kch1)
    ich = (ich0, ich1)
    pch = (pch0, pch1)
    posg = (posg0, posg1)
    offs = (offs0, offs1)
    hloc = (hloc0, hloc1)
    c = lax.axis_index("c")
    t = lax.axis_index("s")
    iota = lax.iota(jnp.int32, L)
    base = t * CHUNK
    zero16 = jnp.zeros((L,), jnp.int32)

    digit_of = (
        lambda v: v & jnp.int32(0x1FF),
        lambda v: lax.shift_right_logical(v, jnp.int32(9)) & jnp.int32(0xFF),
        lambda v: lax.shift_right_logical(v, jnp.int32(17)) & jnp.int32(0xFF),
        lambda v: lax.shift_right_logical(v, jnp.int32(25)) & jnp.int32(0x7F),
    )
    payload_of = (
        None,
        lambda kv, iv: (kv & jnp.int32(-131072)) | iv,
        lambda kv, iv: kv & jnp.int32(-33423361),
        lambda kv, iv: kv & jnp.int32(0x1FFFF),
    )

    def load_phase(p, rows, srcK, srcI):
        if p == 0:
            for u, r in enumerate(rows):
                pltpu.sync_copy(x_hbm.at[r, pl.ds(base, CHUNK)], rawf[u])

            def tl(j, _):
                for u in range(2):
                    b = plsc.bitcast(rawf[u][pl.ds(j * L, L)], jnp.int32)
                    m = lax.shift_right_arithmetic(b, 31)
                    kd = (b ^ (m | jnp.int32(-2147483648))) ^ jnp.int32(-1)
                    kch[u][pl.ds(j * L, L)] = kd
                    ich[u][pl.ds(j * L, L)] = base + j * L + iota
                return 0

            lax.fori_loop(0, NV, tl, 0, unroll=2)
        else:
            hs = []
            for u in range(2):
                hs.append(pltpu.async_copy(
                    srcK.at[pl.ds(u * NP + base, CHUNK)], kch[u], sem))
                if srcI is not None:
                    hs.append(pltpu.async_copy(
                        srcI.at[pl.ds(u * NP + base, CHUNK)], ich[u], sem))
            for h in hs:
                h.wait()

    def hist_phase(p, radix, hgrid):
        dig = digit_of[p]

        def z(j, _):
            hloc[0][pl.ds(j * L, L)] = zero16
            hloc[1][pl.ds(j * L, L)] = zero16
            return 0

        lax.fori_loop(0, radix // L, z, 0)

        def sc_at(u, j):
            d = dig(kch[u][pl.ds(j * L, L)])
            occ, lastm = plsc.scan_count(d)
            return d, occ, lastm

        dA, oA, lA = sc_at(0, 0)
        dB, oB, lB = sc_at(1, 0)

        def hb(j, carry):
            jn = jnp.minimum(j + 1, NV - 1)
            nxt = sc_at(0, jn) + sc_at(1, jn)
            cur = (carry[0:3], carry[3:6])
            for u in range(2):
                d, occ, lastm = cur[u]
                plsc.addupdate_scatter(hloc[u], [d], occ, mask=lastm)
            return nxt

        lax.fori_loop(0, NV, hb, (dA, oA, lA, dB, oB, lB), unroll=2)
        pltpu.sync_copy(hloc[0].at[pl.ds(0, radix)], hgrid.at[t])
        pltpu.sync_copy(hloc[1].at[pl.ds(0, radix)], hgrid.at[NT + t])
        plsc.subcore_barrier()

    def scan_phase(radix, hgrid, gloc):
        for u in range(2):
            pltpu.sync_copy(hgrid.at[pl.ds(u * NT, NT)], gloc)

            def g_body(g, runv):
                part = zero16
                tot = zero16
                for tp in range(NT):
                    rowv = gloc[tp, pl.ds(g * L, L)]
                    part = part + jnp.where(tp < t, rowv, zero16)
                    tot = tot + rowv
                csum = plsc.cumsum(tot)
                offs[u][pl.ds(g * L, L)] = runv + (csum - tot) + part
                return runv + jnp.full((L,), jnp.sum(tot), jnp.int32)

            lax.fori_loop(0, radix // L, g_body, zero16)

    def permute_phase(p, dstK, dstI):
        dig = digit_of[p]
        pld = payload_of[p]

        def sc_at(u, j):
            d = dig(kch[u][pl.ds(j * L, L)])
            occ, lastm = plsc.scan_count(d)
            return d, occ, lastm

        dA, oA, lA = sc_at(0, 0)
        dB, oB, lB = sc_at(1, 0)

        def pb(j, carry):
            jn = jnp.minimum(j + 1, NV - 1)
            nxt = sc_at(0, jn) + sc_at(1, jn)
            cur = (carry[0:3], carry[3:6])
            for u in range(2):
                d, occ, lastm = cur[u]
                bse = plsc.load_gather(offs[u], [d])
                plsc.store_scatter(offs[u], [d], bse + occ, mask=lastm)
                posg[u][j // 8, pl.ds((j % 8) * L, L)] = (
                    bse + occ - 1 + u * NP)
                if pld is not None:
                    kv = kch[u][pl.ds(j * L, L)]
                    iv = ich[u][pl.ds(j * L, L)] if p == 1 else None
                    pch[u][pl.ds(j * L, L)] = pld(kv, iv)
            return nxt

        lax.fori_loop(0, NV, pb, (dA, oA, lA, dB, oB, lB), unroll=2)
        handles = []
        for u in range(2):
            for w in range(NW):
                if p == 0:
                    handles.append(pltpu.async_copy(
                        kch[u].at[pl.ds(w * 128, 128)],
                        dstK.at[posg[u].at[w]], sem))
                    handles.append(pltpu.async_copy(
                        ich[u].at[pl.ds(w * 128, 128)],
                        dstI.at[posg[u].at[w]], sem))
                else:
                    handles.append(pltpu.async_copy(
                        pch[u].at[pl.ds(w * 128, 128)],
                        dstK.at[posg[u].at[w]], sem))
        for h in handles:
            h.wait()
        plsc.subcore_barrier()

    def pair_body(q, _):
        rows = (2 * q * NC + c, (2 * q + 1) * NC + c)
        load_phase(0, rows, None, None)
        hist_phase(0, R0, hg512)
        scan_phase(R0, hg512, gl512)
        permute_phase(0, bufK, bufI)
        load_phase(1, rows, bufK, bufI)
        hist_phase(1, R1, hg256)
        scan_phase(R1, hg256, gl256)
        permute_phase(1, bufP, None)
        load_phase(2, rows, bufP, None)
        hist_phase(2, R2, hg256)
        scan_phase(R2, hg256, gl256)
        permute_phase(2, bufK, None)
        load_phase(3, rows, bufK, None)
        hist_phase(3, R3, hg128)
        scan_phase(R3, hg128, gl128)
        permute_phase(3, bufP, None)
        for u, r in enumerate(rows):
            pltpu.sync_copy(bufP.at[pl.ds(u * NP + base, CHUNK)],
                            out_hbm.at[r, pl.ds(base, CHUNK)])
        plsc.subcore_barrier()
        return 0

    lax.fori_loop(0, PAIRS_PER_SC, pair_body, 0)


def kernel(sort_ip):
    xp = jnp.pad(sort_ip, ((0, 0), (0, NP - N0)),
                 constant_values=-jnp.inf)
    mesh = plsc.VectorSubcoreMesh(core_axis_name="c", subcore_axis_name="s")
    fn = pl.kernel(
        _sort_body,
        out_type=jax.ShapeDtypeStruct((NR, NP), jnp.int32),
        mesh=mesh,
        compiler_params=pltpu.CompilerParams(needs_layout_passes=False),
        scratch_types=(
            pltpu.VMEM_SHARED((2 * NP,), jnp.int32),
            pltpu.VMEM_SHARED((2 * NP,), jnp.int32),
            pltpu.VMEM_SHARED((2 * NP,), jnp.int32),
            pltpu.VMEM_SHARED((2 * NT, R0), jnp.int32),
            pltpu.VMEM_SHARED((2 * NT, R1), jnp.int32),
            pltpu.VMEM_SHARED((2 * NT, R3), jnp.int32),
            pltpu.VMEM((CHUNK,), jnp.float32),
            pltpu.VMEM((CHUNK,), jnp.float32),
            pltpu.VMEM((CHUNK,), jnp.int32),
            pltpu.VMEM((CHUNK,), jnp.int32),
            pltpu.VMEM((CHUNK,), jnp.int32),
            pltpu.VMEM((CHUNK,), jnp.int32),
            pltpu.VMEM((CHUNK,), jnp.int32),
            pltpu.VMEM((CHUNK,), jnp.int32),
            pltpu.VMEM((NW, 128), jnp.int32),
            pltpu.VMEM((NW, 128), jnp.int32),
            pltpu.VMEM((R0,), jnp.int32),
            pltpu.VMEM((R0,), jnp.int32),
            pltpu.VMEM((R0,), jnp.int32),
            pltpu.VMEM((R0,), jnp.int32),
            pltpu.VMEM((NT, R0), jnp.int32),
            pltpu.VMEM((NT, R1), jnp.int32),
            pltpu.VMEM((NT, R3), jnp.int32),
            pltpu.SemaphoreType.DMA,
        ),
    )
    out = fn(xp)
    return out[:, :N0]


# speedup vs baseline: 5.0529x; 1.0126x over previous
import jax
import jax.numpy as jnp
from jax import lax
from jax.experimental import pallas as pl
from jax.experimental.pallas import tpu as pltpu
from jax.experimental.pallas import tpu_sc as plsc

NR = 64
N0 = 100000
L = 16
NT = 16
NC = 2
CHUNK = 6272
NP = NT * CHUNK
NW = CHUNK // 128
NV = CHUNK // L
R0, R1, R2, R3 = 512, 256, 256, 128
PAIRS_PER_SC = NR // NC // 2


def _sort_body(x_hbm, out_hbm, bufK, bufI, bufP, hg512, hg256, hg128,
               rawf0, rawf1, kch0, kch1, ich0, ich1, pch0, pch1,
               posg0, posg1, offs0, offs1, hloc0, hloc1,
               gl512, gl256, gl128, sem):
    rawf = (rawf0, rawf1)
    kch = (kch0, kch1)
    ich = (ich0, ich1)
    pch = (pch0, pch1)
    posg = (posg0, posg1)
    offs = (offs0, offs1)
    hloc = (hloc0, hloc1)
    c = lax.axis_index("c")
    t = lax.axis_index("s")
    iota = lax.iota(jnp.int32, L)
    base = t * CHUNK
    zero16 = jnp.zeros((L,), jnp.int32)

    digit_of = (
        lambda v: v & jnp.int32(0x1FF),
        lambda v: lax.shift_right_logical(v, jnp.int32(9)) & jnp.int32(0xFF),
        lambda v: lax.shift_right_logical(v, jnp.int32(17)) & jnp.int32(0xFF),
        lambda v: lax.shift_right_logical(v, jnp.int32(25)) & jnp.int32(0x7F),
    )
    payload_of = (
        None,
        lambda kv, iv: (kv & jnp.int32(-131072)) | iv,
        lambda kv, iv: kv & jnp.int32(-33423361),
        lambda kv, iv: kv & jnp.int32(0x1FFFF),
    )

    def load_phase(p, rows, srcK, srcI):
        if p == 0:
            for u, r in enumerate(rows):
                pltpu.sync_copy(x_hbm.at[r, pl.ds(base, CHUNK)], rawf[u])

            def tl(j, _):
                for u in range(2):
                    b = plsc.bitcast(rawf[u][pl.ds(j * L, L)], jnp.int32)
                    m = lax.shift_right_arithmetic(b, 31)
                    kd = (b ^ (m | jnp.int32(-2147483648))) ^ jnp.int32(-1)
                    kch[u][pl.ds(j * L, L)] = kd
                    ich[u][pl.ds(j * L, L)] = base + j * L + iota
                return 0

            lax.fori_loop(0, NV, tl, 0, unroll=2)
        else:
            hs = []
            for u in range(2):
                hs.append(pltpu.async_copy(
                    srcK.at[pl.ds(u * NP + base, CHUNK)], kch[u], sem))
                if srcI is not None:
                    hs.append(pltpu.async_copy(
                        srcI.at[pl.ds(u * NP + base, CHUNK)], ich[u], sem))
            for h in hs:
                h.wait()

    def hist_phase(p, radix, hgrid):
        dig = digit_of[p]

        def z(j, _):
            hloc[0][pl.ds(j * L, L)] = zero16
            hloc[1][pl.ds(j * L, L)] = zero16
            return 0

        lax.fori_loop(0, radix // L, z, 0)

        def sc_at(u, j):
            d = dig(kch[u][pl.ds(j * L, L)])
            occ, lastm = plsc.scan_count(d)
            return d, occ, lastm

        dA, oA, lA = sc_at(0, 0)
        dB, oB, lB = sc_at(1, 0)

        def hb(j, carry):
            jn = jnp.minimum(j + 1, NV - 1)
            nxt = sc_at(0, jn) + sc_at(1, jn)
            cur = (carry[0:3], carry[3:6])
            for u in range(2):
                d, occ, lastm = cur[u]
                plsc.addupdate_scatter(hloc[u], [d], occ, mask=lastm)
            return nxt

        lax.fori_loop(0, NV, hb, (dA, oA, lA, dB, oB, lB), unroll=4)
        pltpu.sync_copy(hloc[0].at[pl.ds(0, radix)], hgrid.at[t])
        pltpu.sync_copy(hloc[1].at[pl.ds(0, radix)], hgrid.at[NT + t])
        plsc.subcore_barrier()

    def scan_phase(radix, hgrid, gloc):
        for u in range(2):
            pltpu.sync_copy(hgrid.at[pl.ds(u * NT, NT)], gloc)

            def g_body(g, runv):
                part = zero16
                tot = zero16
                for tp in range(NT):
                    rowv = gloc[tp, pl.ds(g * L, L)]
                    part = part + jnp.where(tp < t, rowv, zero16)
                    tot = tot + rowv
                csum = plsc.cumsum(tot)
                offs[u][pl.ds(g * L, L)] = runv + (csum - tot) + part
                return runv + jnp.full((L,), jnp.sum(tot), jnp.int32)

            lax.fori_loop(0, radix // L, g_body, zero16)

    def permute_phase(p, dstK, dstI):
        dig = digit_of[p]
        pld = payload_of[p]

        def sc_at(u, j):
            d = dig(kch[u][pl.ds(j * L, L)])
            occ, lastm = plsc.scan_count(d)
            return d, occ, lastm

        dA, oA, lA = sc_at(0, 0)
        dB, oB, lB = sc_at(1, 0)

        def pb(j, carry):
            jn = jnp.minimum(j + 1, NV - 1)
            nxt = sc_at(0, jn) + sc_at(1, jn)
            cur = (carry[0:3], carry[3:6])
            for u in range(2):
                d, occ, lastm = cur[u]
                bse = plsc.load_gather(offs[u], [d])
                plsc.store_scatter(offs[u], [d], bse + occ, mask=lastm)
                posg[u][j // 8, pl.ds((j % 8) * L, L)] = (
                    bse + occ - 1 + u * NP)
                if pld is not None:
                    kv = kch[u][pl.ds(j * L, L)]
                    iv = ich[u][pl.ds(j * L, L)] if p == 1 else None
                    pch[u][pl.ds(j * L, L)] = pld(kv, iv)
            return nxt

        lax.fori_loop(0, NV, pb, (dA, oA, lA, dB, oB, lB), unroll=4)
        handles = []
        for u in range(2):
            for w in range(NW):
                if p == 0:
                    handles.append(pltpu.async_copy(
                        kch[u].at[pl.ds(w * 128, 128)],
                        dstK.at[posg[u].at[w]], sem))
                    handles.append(pltpu.async_copy(
                        ich[u].at[pl.ds(w * 128, 128)],
                        dstI.at[posg[u].at[w]], sem))
                else:
                    handles.append(pltpu.async_copy(
                        pch[u].at[pl.ds(w * 128, 128)],
                        dstK.at[posg[u].at[w]], sem))
        for h in handles:
            h.wait()
        plsc.subcore_barrier()

    def pair_body(q, _):
        rows = (2 * q * NC + c, (2 * q + 1) * NC + c)
        load_phase(0, rows, None, None)
        hist_phase(0, R0, hg512)
        scan_phase(R0, hg512, gl512)
        permute_phase(0, bufK, bufI)
        load_phase(1, rows, bufK, bufI)
        hist_phase(1, R1, hg256)
        scan_phase(R1, hg256, gl256)
        permute_phase(1, bufP, None)
        load_phase(2, rows, bufP, None)
        hist_phase(2, R2, hg256)
        scan_phase(R2, hg256, gl256)
        permute_phase(2, bufK, None)
        load_phase(3, rows, bufK, None)
        hist_phase(3, R3, hg128)
        scan_phase(R3, hg128, gl128)
        permute_phase(3, bufP, None)
        for u, r in enumerate(rows):
            pltpu.sync_copy(bufP.at[pl.ds(u * NP + base, CHUNK)],
                            out_hbm.at[r, pl.ds(base, CHUNK)])
        plsc.subcore_barrier()
        return 0

    lax.fori_loop(0, PAIRS_PER_SC, pair_body, 0)


def kernel(sort_ip):
    xp = jnp.pad(sort_ip, ((0, 0), (0, NP - N0)),
                 constant_values=-jnp.inf)
    mesh = plsc.VectorSubcoreMesh(core_axis_name="c", subcore_axis_name="s")
    fn = pl.kernel(
        _sort_body,
        out_type=jax.ShapeDtypeStruct((NR, NP), jnp.int32),
        mesh=mesh,
        compiler_params=pltpu.CompilerParams(needs_layout_passes=False),
        scratch_types=(
            pltpu.VMEM_SHARED((2 * NP,), jnp.int32),
            pltpu.VMEM_SHARED((2 * NP,), jnp.int32),
            pltpu.VMEM_SHARED((2 * NP,), jnp.int32),
            pltpu.VMEM_SHARED((2 * NT, R0), jnp.int32),
            pltpu.VMEM_SHARED((2 * NT, R1), jnp.int32),
            pltpu.VMEM_SHARED((2 * NT, R3), jnp.int32),
            pltpu.VMEM((CHUNK,), jnp.float32),
            pltpu.VMEM((CHUNK,), jnp.float32),
            pltpu.VMEM((CHUNK,), jnp.int32),
            pltpu.VMEM((CHUNK,), jnp.int32),
            pltpu.VMEM((CHUNK,), jnp.int32),
            pltpu.VMEM((CHUNK,), jnp.int32),
            pltpu.VMEM((CHUNK,), jnp.int32),
            pltpu.VMEM((CHUNK,), jnp.int32),
            pltpu.VMEM((NW, 128), jnp.int32),
            pltpu.VMEM((NW, 128), jnp.int32),
            pltpu.VMEM((R0,), jnp.int32),
            pltpu.VMEM((R0,), jnp.int32),
            pltpu.VMEM((R0,), jnp.int32),
            pltpu.VMEM((R0,), jnp.int32),
            pltpu.VMEM((NT, R0), jnp.int32),
            pltpu.VMEM((NT, R1), jnp.int32),
            pltpu.VMEM((NT, R3), jnp.int32),
            pltpu.SemaphoreType.DMA,
        ),
    )
    out = fn(xp)
    return out[:, :N0]
